# baseline (speedup 1.0000x reference)
import functools

import jax
import jax.numpy as jnp
from jax import lax
from jax.experimental import pallas as pl
from jax.experimental.pallas import tpu as pltpu
from jax.experimental.pallas import tpu_sc as plsc

_N_FIELDS = 26
_N_DENSE = 13
_DIM = 32
_EMB_W = _N_FIELDS * _DIM
_OUT_W = _EMB_W + _N_DENSE
_PAD_W = _EMB_W + 16
_CH = 128


def kernel(indices, dense, tables):
    B = indices.shape[1]
    info = plsc.get_sparse_core_info()
    NC, NS = info.num_cores, info.num_subcores
    NW = NC * NS
    rows_per_w = B // NW
    n_chunks = rows_per_w // _CH

    mesh = plsc.VectorSubcoreMesh(core_axis_name="c", subcore_axis_name="s")

    @functools.partial(
        pl.kernel,
        mesh=mesh,
        compiler_params=pltpu.CompilerParams(use_tc_tiling_on_sc=False),
        out_type=jax.ShapeDtypeStruct((B, _PAD_W), jnp.float32),
        scratch_types=[
            pltpu.VMEM((_N_FIELDS, _CH), jnp.int32),
            pltpu.VMEM((_N_FIELDS, _CH, _DIM), jnp.float32),
            pltpu.SemaphoreType.DMA,
            pltpu.SemaphoreType.DMA,
        ],
    )
    def sc_combine(idx_hbm, dense_hbm, tables_hbm, out_hbm,
                   idx_v, tmp_v, gsem, wsem):
        wid = lax.axis_index("s") * NC + lax.axis_index("c")
        base = wid * rows_per_w

        dense_copies = [
            pltpu.async_copy(
                dense_hbm.at[pl.ds(base, rows_per_w), :],
                out_hbm.at[pl.ds(base, rows_per_w), pl.ds(_EMB_W, 16)],
                wsem),
        ]

        def chunk_body(c, carry):
            rowbase = base + c * _CH
            pltpu.sync_copy(idx_hbm.at[:, pl.ds(rowbase, _CH)], idx_v)
            gathers = [
                pltpu.async_copy(tables_hbm.at[f].at[idx_v.at[f]],
                                 tmp_v.at[f], gsem)
                for f in range(_N_FIELDS)
            ]
            writes = []
            for f in range(_N_FIELDS):
                gathers[f].wait()
                writes.append(pltpu.async_copy(
                    tmp_v.at[f],
                    out_hbm.at[pl.ds(rowbase, _CH), pl.ds(f * _DIM, _DIM)],
                    wsem))
            for w in writes:
                w.wait()
            return carry

        lax.fori_loop(0, n_chunks, chunk_body, None)
        for cp in dense_copies:
            cp.wait()

    dense_t = jnp.pad(jnp.transpose(dense), ((0, 0), (0, 3)))
    return sc_combine(indices, dense_t, tables)[:, :_OUT_W]
